# baseline (speedup 1.0000x reference)
import jax
import jax.numpy as jnp
from jax import lax
from jax.experimental import pallas as pl
from jax.experimental.pallas import tpu as pltpu
from jax.experimental.pallas import tpu_sc as plsc

N = 10000
E = 320000
DF = 8
ACCW = 9
NC = 2
NS = 16
NW = NC * NS
EPW = E // NW
BR = 320
CAPL = 512

_SC_PARAMS = pltpu.CompilerParams(
    needs_layout_passes=False, use_tc_tiling_on_sc=False)
_SC_MESH = plsc.VectorSubcoreMesh(core_axis_name="c", subcore_axis_name="s")


def _bucket_body(src_hbm, dst_hbm, bsrc_hbm, bdst_hbm, cnt_hbm,
                 esrc, edst, bbs, bbd, cur, slot):
    c = lax.axis_index("c")
    s = lax.axis_index("s")
    wid = c * NS + s

    pltpu.sync_copy(src_hbm.at[pl.ds(wid * EPW, EPW)], esrc)
    pltpu.sync_copy(dst_hbm.at[pl.ds(wid * EPW, EPW)], edst)

    lanes = lax.iota(jnp.int32, 16)
    zero16 = jnp.zeros((16,), jnp.int32)
    cur[pl.ds(0, 16)] = zero16
    cur[pl.ds(16, 16)] = zero16

    def vblk(v, carry):
        off = v * 16
        sv = esrc[pl.ds(off, 16)]
        dv = edst[pl.ds(off, 16)]
        bv = dv // BR

        def not_done(state):
            return plsc.all_reduce_population_count(state[0] > 0)[0] > 0

        def round_(state):
            active = state[0] > 0
            plsc.store_scatter(slot, [bv], lanes, mask=active)
            back = plsc.load_gather(slot, [bv])
            win = active & (back == lanes)
            kv = plsc.load_gather(cur, [bv])
            kv = jnp.minimum(kv, CAPL - 1)
            plsc.store_scatter(bbs, [bv, kv], sv, mask=win)
            plsc.store_scatter(bbd, [bv, kv], dv, mask=win)
            plsc.store_scatter(cur, [bv], kv + 1, mask=win)
            return (jnp.where(win, 0, state[0]),)

        lax.while_loop(not_done, round_, (jnp.ones((16,), jnp.int32),))
        return carry

    lax.fori_loop(0, EPW // 16, vblk, 0)

    pltpu.sync_copy(bbs, bsrc_hbm.at[wid])
    pltpu.sync_copy(bbd, bdst_hbm.at[wid])
    pltpu.sync_copy(cur, cnt_hbm.at[wid])


_bucketize = pl.kernel(
    _bucket_body,
    out_type=(
        jax.ShapeDtypeStruct((NW, NW, CAPL), jnp.int32),
        jax.ShapeDtypeStruct((NW, NW, CAPL), jnp.int32),
        jax.ShapeDtypeStruct((NW, NW), jnp.int32),
    ),
    mesh=_SC_MESH,
    compiler_params=_SC_PARAMS,
    scratch_types=[
        pltpu.VMEM((EPW,), jnp.int32),
        pltpu.VMEM((EPW,), jnp.int32),
        pltpu.VMEM((NW, CAPL), jnp.int32),
        pltpu.VMEM((NW, CAPL), jnp.int32),
        pltpu.VMEM((NW,), jnp.int32),
        pltpu.VMEM((NW,), jnp.int32),
    ],
)


def _edge_body(bsrc_hbm, bdst_hbm, cnt_hbm, asad_hbm, xp_hbm, zeros_hbm,
               out_hbm, asad_v, xp_v, cnt_v, ss0, sd0, ss1, sd1, accl,
               e0, e1):
    c = lax.axis_index("c")
    s = lax.axis_index("s")
    t = c * NS + s
    base = t * BR

    pltpu.sync_copy(zeros_hbm, accl)
    pltpu.sync_copy(asad_hbm, asad_v)
    pltpu.sync_copy(xp_hbm, xp_v)
    pltpu.sync_copy(cnt_hbm.at[t], cnt_v)

    lanes = lax.iota(jnp.int32, 16)
    bufs = ((ss0, sd0, e0), (ss1, sd1, e1))

    def start_seg(p, ss, sd, sem):
        pltpu.async_copy(bsrc_hbm.at[p, t], ss, sem)
        pltpu.async_copy(bdst_hbm.at[p, t], sd, sem)

    def wait_seg(ss, sd, sem):
        pltpu.make_async_copy(bsrc_hbm.at[0, 0], ss, sem).wait()
        pltpu.make_async_copy(bdst_hbm.at[0, 0], sd, sem).wait()

    start_seg(0, ss0, sd0, e0)
    start_seg(1, ss1, sd1, e1)

    def seg_pair(i2, carry):
        for b in range(2):
            p = 2 * i2 + b
            ss, sd, sem = bufs[b]
            wait_seg(ss, sd, sem)
            cnt = plsc.load_gather(cnt_v, [jnp.full((16,), p, jnp.int32)])[0]
            nv = (cnt + 15) // 16

            @plsc.parallel_loop(0, nv, unroll=4)
            def vb(v, ss=ss, sd=sd, cnt=cnt):
                off = v * 16
                m = (lanes + off) < cnt
                sv = jnp.where(m, ss[pl.ds(off, 16)], 0)
                dv = jnp.where(m, sd[pl.ds(off, 16)], base)
                a_s = plsc.load_gather(asad_v, [sv])
                a_d = plsc.load_gather(asad_v, [dv + N])
                e = a_s + a_d
                e = jnp.where(e >= 0.0, e, 0.2 * e)
                g = jnp.where(m, jnp.exp(e), 0.0)
                dvr = dv - base
                plsc.addupdate_scatter(accl, [dvr], g)
                for f in range(DF):
                    xf = plsc.load_gather(xp_v, [sv + f * N])
                    plsc.addupdate_scatter(accl, [dvr + (f + 1) * BR],
                                           g * xf)

            @pl.when(p + 2 < NW)
            def _():
                start_seg(p + 2, ss, sd, sem)

        return carry

    lax.fori_loop(0, NW // 2, seg_pair, 0)

    pltpu.sync_copy(accl, out_hbm.at[t])


_edge_pass = pl.kernel(
    _edge_body,
    out_type=jax.ShapeDtypeStruct((NW, ACCW * BR), jnp.float32),
    mesh=_SC_MESH,
    compiler_params=_SC_PARAMS,
    scratch_types=[
        pltpu.VMEM((N * 2,), jnp.float32),
        pltpu.VMEM((N * DF,), jnp.float32),
        pltpu.VMEM((NW,), jnp.int32),
        pltpu.VMEM((CAPL,), jnp.int32),
        pltpu.VMEM((CAPL,), jnp.int32),
        pltpu.VMEM((CAPL,), jnp.int32),
        pltpu.VMEM((CAPL,), jnp.int32),
        pltpu.VMEM((ACCW * BR,), jnp.float32),
        pltpu.SemaphoreType.DMA,
        pltpu.SemaphoreType.DMA,
    ],
)


def _prep_body(xt_ref, wt_ref, asr_ref, adr_ref, xp_ref, asad_ref):
    xpt = jnp.dot(wt_ref[...], xt_ref[...],
                  preferred_element_type=jnp.float32)
    xp_ref[...] = xpt
    a_s = jnp.dot(asr_ref[...], xpt, preferred_element_type=jnp.float32)
    a_d = jnp.dot(adr_ref[...], xpt, preferred_element_type=jnp.float32)
    asad_ref[...] = jnp.concatenate([a_s, a_d], axis=0)


def _tc_prep(ht, wt, a_src, a_dst):
    return pl.pallas_call(
        _prep_body,
        out_shape=[
            jax.ShapeDtypeStruct((DF, N), jnp.float32),
            jax.ShapeDtypeStruct((2, N), jnp.float32),
        ],
    )(ht, wt, a_src, a_dst)


def _mid_body(p_ref, b_ref, wt_ref, asr_ref, adr_ref, xp_ref, asad_ref):
    p = p_ref[...]
    denom = p[0:1, :]
    feats = p[1:1 + DF, :]
    h = jnp.maximum(feats / (denom + 1e-16) + b_ref[...], 0.0)
    xpt = jnp.dot(wt_ref[...], h, preferred_element_type=jnp.float32)
    xp_ref[...] = xpt
    a_s = jnp.dot(asr_ref[...], xpt, preferred_element_type=jnp.float32)
    a_d = jnp.dot(adr_ref[...], xpt, preferred_element_type=jnp.float32)
    asad_ref[...] = jnp.concatenate([a_s, a_d], axis=0)


def _tc_mid(acct, b, wt, a_src, a_dst):
    return pl.pallas_call(
        _mid_body,
        out_shape=[
            jax.ShapeDtypeStruct((DF, N), jnp.float32),
            jax.ShapeDtypeStruct((2, N), jnp.float32),
        ],
    )(acct, b, wt, a_src, a_dst)


def _final_body(p_ref, b_ref, out_ref):
    p = p_ref[...]
    out_ref[...] = jax.nn.sigmoid(
        p[1:2, :] / (p[0:1, :] + 1e-16) + b_ref[...])


def _tc_final(acct, b):
    return pl.pallas_call(
        _final_body,
        out_shape=jax.ShapeDtypeStruct((1, N), jnp.float32),
    )(acct, b)


def kernel(x, edge_index, W1, a_src1, a_dst1, b1, W2, a_src2, a_dst2, b2,
           W3, a_src3, a_dst3, b3):
    src = edge_index[0]
    dst = edge_index[1]
    zeros = jnp.zeros((ACCW * BR,), jnp.float32)

    W3p = jnp.pad(W3, ((0, 0), (0, DF - W3.shape[1])))
    a_src3p = jnp.pad(a_src3, (0, DF - a_src3.shape[0]))
    a_dst3p = jnp.pad(a_dst3, (0, DF - a_dst3.shape[0]))

    bsrc, bdst, cnts = _bucketize(src, dst)
    cntsT = cnts.T

    def edge_pass(asadt, xpt):
        out = _edge_pass(bsrc, bdst, cntsT, asadt.reshape(-1),
                         xpt.reshape(-1), zeros)
        return out.reshape(NW, ACCW, BR).transpose(1, 0, 2).reshape(
            ACCW, NW * BR)[:, :N]

    xp1, asad1 = _tc_prep(x.T, W1.T, a_src1.reshape(1, DF),
                          a_dst1.reshape(1, DF))
    acc1 = edge_pass(asad1, xp1)
    xp2, asad2 = _tc_mid(acc1, b1.reshape(DF, 1), W2.T,
                         a_src2.reshape(1, DF), a_dst2.reshape(1, DF))
    acc2 = edge_pass(asad2, xp2)
    xp3, asad3 = _tc_mid(acc2, b2.reshape(DF, 1), W3p.T,
                         a_src3p.reshape(1, DF), a_dst3p.reshape(1, DF))
    acc3 = edge_pass(asad3, xp3)
    return _tc_final(acc3, b3.reshape(1, 1)).reshape(N, 1)


# speedup vs baseline: 1.1127x; 1.1127x over previous
import jax
import jax.numpy as jnp
from jax import lax
from jax.experimental import pallas as pl
from jax.experimental.pallas import tpu as pltpu
from jax.experimental.pallas import tpu_sc as plsc

N = 10000
E = 320000
DF = 8
ACCW = 9
NC = 2
NS = 16
NW = NC * NS
EPW = E // NW
BR = 320
CAPL = 512

_SC_PARAMS = pltpu.CompilerParams(
    needs_layout_passes=False, use_tc_tiling_on_sc=False)
_SC_MESH = plsc.VectorSubcoreMesh(core_axis_name="c", subcore_axis_name="s")


def _bucket_body(src_hbm, dst_hbm, bsrc_hbm, bdst_hbm, cnt_hbm,
                 esrc, edst, bbs, bbd, cur):
    c = lax.axis_index("c")
    s = lax.axis_index("s")
    wid = c * NS + s

    pltpu.sync_copy(src_hbm.at[pl.ds(wid * EPW, EPW)], esrc)
    pltpu.sync_copy(dst_hbm.at[pl.ds(wid * EPW, EPW)], edst)

    zero16 = jnp.zeros((16,), jnp.int32)
    cur[pl.ds(0, 16)] = zero16
    cur[pl.ds(16, 16)] = zero16

    def vblk(v, carry):
        off = v * 16
        sv = esrc[pl.ds(off, 16)]
        dv = edst[pl.ds(off, 16)]
        bv = dv // BR
        cnt16, lastm = plsc.scan_count(bv)
        kv = plsc.load_gather(cur, [bv])
        pos = jnp.minimum(kv + cnt16 - 1, CAPL - 1)
        plsc.store_scatter(bbs, [bv, pos], sv)
        plsc.store_scatter(bbd, [bv, pos], dv)
        plsc.store_scatter(cur, [bv], jnp.minimum(kv + cnt16, CAPL),
                           mask=lastm)
        return carry

    lax.fori_loop(0, EPW // 16, vblk, 0)

    pltpu.sync_copy(bbs, bsrc_hbm.at[wid])
    pltpu.sync_copy(bbd, bdst_hbm.at[wid])
    pltpu.sync_copy(cur, cnt_hbm.at[wid])


_bucketize = pl.kernel(
    _bucket_body,
    out_type=(
        jax.ShapeDtypeStruct((NW, NW, CAPL), jnp.int32),
        jax.ShapeDtypeStruct((NW, NW, CAPL), jnp.int32),
        jax.ShapeDtypeStruct((NW, NW), jnp.int32),
    ),
    mesh=_SC_MESH,
    compiler_params=_SC_PARAMS,
    scratch_types=[
        pltpu.VMEM((EPW,), jnp.int32),
        pltpu.VMEM((EPW,), jnp.int32),
        pltpu.VMEM((NW, CAPL), jnp.int32),
        pltpu.VMEM((NW, CAPL), jnp.int32),
        pltpu.VMEM((NW,), jnp.int32),
    ],
)


def _edge_body(bsrc_hbm, bdst_hbm, cnt_hbm, asad_hbm, xp_hbm, zeros_hbm,
               out_hbm, asad_v, xp_v, cnt_v, ss0, sd0, ss1, sd1, accl,
               e0, e1):
    c = lax.axis_index("c")
    s = lax.axis_index("s")
    t = c * NS + s
    base = t * BR

    pltpu.sync_copy(zeros_hbm, accl)
    pltpu.sync_copy(asad_hbm, asad_v)
    pltpu.sync_copy(xp_hbm, xp_v)
    pltpu.sync_copy(cnt_hbm.at[t], cnt_v)

    lanes = lax.iota(jnp.int32, 16)
    bufs = ((ss0, sd0, e0), (ss1, sd1, e1))

    def start_seg(p, ss, sd, sem):
        pltpu.async_copy(bsrc_hbm.at[p, t], ss, sem)
        pltpu.async_copy(bdst_hbm.at[p, t], sd, sem)

    def wait_seg(ss, sd, sem):
        pltpu.make_async_copy(bsrc_hbm.at[0, 0], ss, sem).wait()
        pltpu.make_async_copy(bdst_hbm.at[0, 0], sd, sem).wait()

    start_seg(0, ss0, sd0, e0)
    start_seg(1, ss1, sd1, e1)

    def seg_pair(i2, carry):
        for b in range(2):
            p = 2 * i2 + b
            ss, sd, sem = bufs[b]
            wait_seg(ss, sd, sem)
            cnt = plsc.load_gather(cnt_v, [jnp.full((16,), p, jnp.int32)])[0]
            nv = (cnt + 15) // 16

            @plsc.parallel_loop(0, nv, unroll=2)
            def vb(v, ss=ss, sd=sd, cnt=cnt):
                off = v * 16
                m = (lanes + off) < cnt
                sv = jnp.where(m, ss[pl.ds(off, 16)], 0)
                dv = jnp.where(m, sd[pl.ds(off, 16)], base)
                a_s = plsc.load_gather(asad_v, [sv])
                a_d = plsc.load_gather(asad_v, [dv + N])
                e = a_s + a_d
                e = jnp.where(e >= 0.0, e, 0.2 * e)
                g = jnp.where(m, jnp.exp(e), 0.0)
                dvr = dv - base
                plsc.addupdate_scatter(accl, [dvr], g)
                for f in range(DF):
                    xf = plsc.load_gather(xp_v, [sv + f * N])
                    plsc.addupdate_scatter(accl, [dvr + (f + 1) * BR],
                                           g * xf)

            @pl.when(p + 2 < NW)
            def _():
                start_seg(p + 2, ss, sd, sem)

        return carry

    lax.fori_loop(0, NW // 2, seg_pair, 0)

    pltpu.sync_copy(accl, out_hbm.at[t])


_edge_pass = pl.kernel(
    _edge_body,
    out_type=jax.ShapeDtypeStruct((NW, ACCW * BR), jnp.float32),
    mesh=_SC_MESH,
    compiler_params=_SC_PARAMS,
    scratch_types=[
        pltpu.VMEM((N * 2,), jnp.float32),
        pltpu.VMEM((N * DF,), jnp.float32),
        pltpu.VMEM((NW,), jnp.int32),
        pltpu.VMEM((CAPL,), jnp.int32),
        pltpu.VMEM((CAPL,), jnp.int32),
        pltpu.VMEM((CAPL,), jnp.int32),
        pltpu.VMEM((CAPL,), jnp.int32),
        pltpu.VMEM((ACCW * BR,), jnp.float32),
        pltpu.SemaphoreType.DMA,
        pltpu.SemaphoreType.DMA,
    ],
)


def _prep_body(xt_ref, wt_ref, asr_ref, adr_ref, xp_ref, asad_ref):
    xpt = jnp.dot(wt_ref[...], xt_ref[...],
                  preferred_element_type=jnp.float32)
    xp_ref[...] = xpt
    a_s = jnp.dot(asr_ref[...], xpt, preferred_element_type=jnp.float32)
    a_d = jnp.dot(adr_ref[...], xpt, preferred_element_type=jnp.float32)
    asad_ref[...] = jnp.concatenate([a_s, a_d], axis=0)


def _tc_prep(ht, wt, a_src, a_dst):
    return pl.pallas_call(
        _prep_body,
        out_shape=[
            jax.ShapeDtypeStruct((DF, N), jnp.float32),
            jax.ShapeDtypeStruct((2, N), jnp.float32),
        ],
    )(ht, wt, a_src, a_dst)


def _mid_body(p_ref, b_ref, wt_ref, asr_ref, adr_ref, xp_ref, asad_ref):
    p = p_ref[...]
    denom = p[0:1, :]
    feats = p[1:1 + DF, :]
    h = jnp.maximum(feats / (denom + 1e-16) + b_ref[...], 0.0)
    xpt = jnp.dot(wt_ref[...], h, preferred_element_type=jnp.float32)
    xp_ref[...] = xpt
    a_s = jnp.dot(asr_ref[...], xpt, preferred_element_type=jnp.float32)
    a_d = jnp.dot(adr_ref[...], xpt, preferred_element_type=jnp.float32)
    asad_ref[...] = jnp.concatenate([a_s, a_d], axis=0)


def _tc_mid(acct, b, wt, a_src, a_dst):
    return pl.pallas_call(
        _mid_body,
        out_shape=[
            jax.ShapeDtypeStruct((DF, N), jnp.float32),
            jax.ShapeDtypeStruct((2, N), jnp.float32),
        ],
    )(acct, b, wt, a_src, a_dst)


def _final_body(p_ref, b_ref, out_ref):
    p = p_ref[...]
    out_ref[...] = jax.nn.sigmoid(
        p[1:2, :] / (p[0:1, :] + 1e-16) + b_ref[...])


def _tc_final(acct, b):
    return pl.pallas_call(
        _final_body,
        out_shape=jax.ShapeDtypeStruct((1, N), jnp.float32),
    )(acct, b)


def kernel(x, edge_index, W1, a_src1, a_dst1, b1, W2, a_src2, a_dst2, b2,
           W3, a_src3, a_dst3, b3):
    src = edge_index[0]
    dst = edge_index[1]
    zeros = jnp.zeros((ACCW * BR,), jnp.float32)

    W3p = jnp.pad(W3, ((0, 0), (0, DF - W3.shape[1])))
    a_src3p = jnp.pad(a_src3, (0, DF - a_src3.shape[0]))
    a_dst3p = jnp.pad(a_dst3, (0, DF - a_dst3.shape[0]))

    bsrc, bdst, cnts = _bucketize(src, dst)
    cntsT = cnts.T

    def edge_pass(asadt, xpt):
        out = _edge_pass(bsrc, bdst, cntsT, asadt.reshape(-1),
                         xpt.reshape(-1), zeros)
        return out.reshape(NW, ACCW, BR).transpose(1, 0, 2).reshape(
            ACCW, NW * BR)[:, :N]

    xp1, asad1 = _tc_prep(x.T, W1.T, a_src1.reshape(1, DF),
                          a_dst1.reshape(1, DF))
    acc1 = edge_pass(asad1, xp1)
    xp2, asad2 = _tc_mid(acc1, b1.reshape(DF, 1), W2.T,
                         a_src2.reshape(1, DF), a_dst2.reshape(1, DF))
    acc2 = edge_pass(asad2, xp2)
    xp3, asad3 = _tc_mid(acc2, b2.reshape(DF, 1), W3p.T,
                         a_src3p.reshape(1, DF), a_dst3p.reshape(1, DF))
    acc3 = edge_pass(asad3, xp3)
    return _tc_final(acc3, b3.reshape(1, 1)).reshape(N, 1)


# speedup vs baseline: 1.1206x; 1.0071x over previous
import jax
import jax.numpy as jnp
from jax import lax
from jax.experimental import pallas as pl
from jax.experimental.pallas import tpu as pltpu
from jax.experimental.pallas import tpu_sc as plsc

N = 10000
E = 320000
DF = 8
ACCW = 9
NC = 2
NS = 16
NW = NC * NS
EPW = E // NW
BR = 320
CAPL = 512

_SC_PARAMS = pltpu.CompilerParams(
    needs_layout_passes=False, use_tc_tiling_on_sc=False)
_SC_MESH = plsc.VectorSubcoreMesh(core_axis_name="c", subcore_axis_name="s")


def _bucket_body(src_hbm, dst_hbm, bsrc_hbm, bdst_hbm, cnt_hbm,
                 esrc, edst, bbs, bbd, cur, tsem):
    c = lax.axis_index("c")
    s = lax.axis_index("s")
    wid = c * NS + s

    t1 = pltpu.async_copy(src_hbm.at[pl.ds(wid * EPW, EPW)], esrc, tsem)
    t2 = pltpu.async_copy(dst_hbm.at[pl.ds(wid * EPW, EPW)], edst, tsem)
    t1.wait()
    t2.wait()

    zero16 = jnp.zeros((16,), jnp.int32)
    cur[pl.ds(0, 16)] = zero16
    cur[pl.ds(16, 16)] = zero16

    def vblk(v, carry):
        off = v * 16
        sv = esrc[pl.ds(off, 16)]
        dv = edst[pl.ds(off, 16)]
        bv = dv // BR
        cnt16, lastm = plsc.scan_count(bv)
        kv = plsc.load_gather(cur, [bv])
        pos = jnp.minimum(kv + cnt16 - 1, CAPL - 1)
        plsc.store_scatter(bbs, [bv, pos], sv)
        plsc.store_scatter(bbd, [bv, pos], dv)
        plsc.store_scatter(cur, [bv], jnp.minimum(kv + cnt16, CAPL),
                           mask=lastm)
        return carry

    lax.fori_loop(0, EPW // 16, vblk, 0)

    pltpu.sync_copy(bbs, bsrc_hbm.at[wid])
    pltpu.sync_copy(bbd, bdst_hbm.at[wid])
    pltpu.sync_copy(cur, cnt_hbm.at[wid])


_bucketize = pl.kernel(
    _bucket_body,
    out_type=(
        jax.ShapeDtypeStruct((NW, NW, CAPL), jnp.int32),
        jax.ShapeDtypeStruct((NW, NW, CAPL), jnp.int32),
        jax.ShapeDtypeStruct((NW, NW), jnp.int32),
    ),
    mesh=_SC_MESH,
    compiler_params=_SC_PARAMS,
    scratch_types=[
        pltpu.VMEM((EPW,), jnp.int32),
        pltpu.VMEM((EPW,), jnp.int32),
        pltpu.VMEM((NW, CAPL), jnp.int32),
        pltpu.VMEM((NW, CAPL), jnp.int32),
        pltpu.VMEM((NW,), jnp.int32),
        pltpu.SemaphoreType.DMA,
    ],
)


def _edge_body(bsrc_hbm, bdst_hbm, cnt_hbm, asad_hbm, xp_hbm, zeros_hbm,
               out_hbm, asad_v, xp_v, cnt_v, ss0, sd0, ss1, sd1, accl,
               e0, e1, tsem):
    c = lax.axis_index("c")
    s = lax.axis_index("s")
    t = c * NS + s
    base = t * BR

    lanes = lax.iota(jnp.int32, 16)
    bufs = ((ss0, sd0, e0), (ss1, sd1, e1))

    def start_seg(p, ss, sd, sem):
        pltpu.async_copy(bsrc_hbm.at[p, t], ss, sem)
        pltpu.async_copy(bdst_hbm.at[p, t], sd, sem)

    def wait_seg(ss, sd, sem):
        pltpu.make_async_copy(bsrc_hbm.at[0, 0], ss, sem).wait()
        pltpu.make_async_copy(bdst_hbm.at[0, 0], sd, sem).wait()

    tz = pltpu.async_copy(zeros_hbm, accl, tsem)
    ta = pltpu.async_copy(asad_hbm, asad_v, tsem)
    tx = pltpu.async_copy(xp_hbm, xp_v, tsem)
    tc_ = pltpu.async_copy(cnt_hbm.at[t], cnt_v, tsem)
    start_seg(0, ss0, sd0, e0)
    start_seg(1, ss1, sd1, e1)
    tz.wait()
    ta.wait()
    tx.wait()
    tc_.wait()

    def seg_pair(i2, carry):
        for b in range(2):
            p = 2 * i2 + b
            ss, sd, sem = bufs[b]
            wait_seg(ss, sd, sem)
            cnt = plsc.load_gather(cnt_v, [jnp.full((16,), p, jnp.int32)])[0]
            nv = (cnt + 15) // 16

            @plsc.parallel_loop(0, nv, unroll=2)
            def vb(v, ss=ss, sd=sd, cnt=cnt):
                off = v * 16
                m = (lanes + off) < cnt
                sv = jnp.where(m, ss[pl.ds(off, 16)], 0)
                dv = jnp.where(m, sd[pl.ds(off, 16)], base)
                a_s = plsc.load_gather(asad_v, [sv])
                a_d = plsc.load_gather(asad_v, [dv + N])
                e = a_s + a_d
                e = jnp.where(e >= 0.0, e, 0.2 * e)
                g = jnp.where(m, jnp.exp(e), 0.0)
                dvr = dv - base
                plsc.addupdate_scatter(accl, [dvr], g)
                for f in range(DF):
                    xf = plsc.load_gather(xp_v, [sv + f * N])
                    plsc.addupdate_scatter(accl, [dvr + (f + 1) * BR],
                                           g * xf)

            @pl.when(p + 2 < NW)
            def _():
                start_seg(p + 2, ss, sd, sem)

        return carry

    lax.fori_loop(0, NW // 2, seg_pair, 0)

    pltpu.sync_copy(accl, out_hbm.at[t])


_edge_pass = pl.kernel(
    _edge_body,
    out_type=jax.ShapeDtypeStruct((NW, ACCW * BR), jnp.float32),
    mesh=_SC_MESH,
    compiler_params=_SC_PARAMS,
    scratch_types=[
        pltpu.VMEM((N * 2,), jnp.float32),
        pltpu.VMEM((N * DF,), jnp.float32),
        pltpu.VMEM((NW,), jnp.int32),
        pltpu.VMEM((CAPL,), jnp.int32),
        pltpu.VMEM((CAPL,), jnp.int32),
        pltpu.VMEM((CAPL,), jnp.int32),
        pltpu.VMEM((CAPL,), jnp.int32),
        pltpu.VMEM((ACCW * BR,), jnp.float32),
        pltpu.SemaphoreType.DMA,
        pltpu.SemaphoreType.DMA,
        pltpu.SemaphoreType.DMA,
    ],
)


def _prep_body(xt_ref, wt_ref, asr_ref, adr_ref, xp_ref, asad_ref):
    xpt = jnp.dot(wt_ref[...], xt_ref[...],
                  preferred_element_type=jnp.float32)
    xp_ref[...] = xpt
    a_s = jnp.dot(asr_ref[...], xpt, preferred_element_type=jnp.float32)
    a_d = jnp.dot(adr_ref[...], xpt, preferred_element_type=jnp.float32)
    asad_ref[...] = jnp.concatenate([a_s, a_d], axis=0)


def _tc_prep(ht, wt, a_src, a_dst):
    return pl.pallas_call(
        _prep_body,
        out_shape=[
            jax.ShapeDtypeStruct((DF, N), jnp.float32),
            jax.ShapeDtypeStruct((2, N), jnp.float32),
        ],
    )(ht, wt, a_src, a_dst)


def _mid_body(p_ref, b_ref, wt_ref, asr_ref, adr_ref, xp_ref, asad_ref):
    p = p_ref[...]
    denom = p[0:1, :]
    feats = p[1:1 + DF, :]
    h = jnp.maximum(feats / (denom + 1e-16) + b_ref[...], 0.0)
    xpt = jnp.dot(wt_ref[...], h, preferred_element_type=jnp.float32)
    xp_ref[...] = xpt
    a_s = jnp.dot(asr_ref[...], xpt, preferred_element_type=jnp.float32)
    a_d = jnp.dot(adr_ref[...], xpt, preferred_element_type=jnp.float32)
    asad_ref[...] = jnp.concatenate([a_s, a_d], axis=0)


def _tc_mid(acct, b, wt, a_src, a_dst):
    return pl.pallas_call(
        _mid_body,
        out_shape=[
            jax.ShapeDtypeStruct((DF, N), jnp.float32),
            jax.ShapeDtypeStruct((2, N), jnp.float32),
        ],
    )(acct, b, wt, a_src, a_dst)


def _final_body(p_ref, b_ref, out_ref):
    p = p_ref[...]
    out_ref[...] = jax.nn.sigmoid(
        p[1:2, :] / (p[0:1, :] + 1e-16) + b_ref[...])


def _tc_final(acct, b):
    return pl.pallas_call(
        _final_body,
        out_shape=jax.ShapeDtypeStruct((1, N), jnp.float32),
    )(acct, b)


def kernel(x, edge_index, W1, a_src1, a_dst1, b1, W2, a_src2, a_dst2, b2,
           W3, a_src3, a_dst3, b3):
    src = edge_index[0]
    dst = edge_index[1]
    zeros = jnp.zeros((ACCW * BR,), jnp.float32)

    W3p = jnp.pad(W3, ((0, 0), (0, DF - W3.shape[1])))
    a_src3p = jnp.pad(a_src3, (0, DF - a_src3.shape[0]))
    a_dst3p = jnp.pad(a_dst3, (0, DF - a_dst3.shape[0]))

    bsrc, bdst, cnts = _bucketize(src, dst)
    cntsT = cnts.T

    def edge_pass(asadt, xpt):
        out = _edge_pass(bsrc, bdst, cntsT, asadt.reshape(-1),
                         xpt.reshape(-1), zeros)
        return out.reshape(NW, ACCW, BR).transpose(1, 0, 2).reshape(
            ACCW, NW * BR)[:, :N]

    xp1, asad1 = _tc_prep(x.T, W1.T, a_src1.reshape(1, DF),
                          a_dst1.reshape(1, DF))
    acc1 = edge_pass(asad1, xp1)
    xp2, asad2 = _tc_mid(acc1, b1.reshape(DF, 1), W2.T,
                         a_src2.reshape(1, DF), a_dst2.reshape(1, DF))
    acc2 = edge_pass(asad2, xp2)
    xp3, asad3 = _tc_mid(acc2, b2.reshape(DF, 1), W3p.T,
                         a_src3p.reshape(1, DF), a_dst3p.reshape(1, DF))
    acc3 = edge_pass(asad3, xp3)
    return _tc_final(acc3, b3.reshape(1, 1)).reshape(N, 1)
